# Initial kernel scaffold; baseline (speedup 1.0000x reference)
import jax
import jax.numpy as jnp
from jax.experimental import pallas as pl


def kernel(x, adj_mat, W_conv, b_conv, W2, b2):
    raise NotImplementedError("write your pallas kernel here")


# speedup vs baseline: 3.0840x; 3.0840x over previous
import functools

import jax
import jax.numpy as jnp
from jax import lax
from jax.experimental import pallas as pl
from jax.experimental.pallas import tpu as pltpu
from jax.experimental.pallas import tpu_sc as plsc

B, N, C, K = 8, 2048, 256, 16
T = K + 1
NODES = B * N
NC, NS, L = 2, 16, 16
NW = NC * NS
NPW = NODES // NW
G = 8
BM = 512
BN = 256


def _fold_body(a_ref, w2t_ref, bc_ref, b2_ref, m_ref, bias_ref):
    m_ref[...] = jnp.dot(a_ref[...], w2t_ref[...],
                         preferred_element_type=jnp.float32)
    bias_ref[...] = jnp.dot(bc_ref[...], w2t_ref[...],
                            preferred_element_type=jnp.float32) + b2_ref[...]


def _fold(acat, w2t, bc, b2):
    return pl.pallas_call(
        _fold_body,
        out_shape=(jax.ShapeDtypeStruct((T * C, C), jnp.float32),
                   jax.ShapeDtypeStruct((1, C), jnp.float32)),
    )(acat, w2t, bc, b2)


def _proj_body(x_ref, m_ref, bias_ref, z_ref):
    t = pl.program_id(1)
    acc = jnp.dot(x_ref[...], m_ref[0],
                  preferred_element_type=jnp.float32)
    z_ref[...] = jnp.where(t == 0, acc + bias_ref[...], acc)


def _proj(x2, mstack, bias):
    nmb = N // BM
    return pl.pallas_call(
        _proj_body,
        grid=(B, T, nmb),
        in_specs=[
            pl.BlockSpec((BM, C), lambda b, t, mb: (b * nmb + mb, 0)),
            pl.BlockSpec((1, C, C), lambda b, t, mb: (t, 0, 0)),
            pl.BlockSpec((1, C), lambda b, t, mb: (0, 0)),
        ],
        out_specs=pl.BlockSpec(
            (BM, C), lambda b, t, mb: ((b * T + t) * nmb + mb, 0)),
        out_shape=jax.ShapeDtypeStruct((B * T * N, C), jnp.float32),
    )(x2, mstack, bias)


def _idx_body(adj_ref, gidx_ref):
    b = pl.program_id(0)
    a = adj_ref[...]
    k_iota = lax.broadcasted_iota(jnp.int32, (BN, K), 1)
    gidx_ref[...] = (b * T + 1 + k_iota) * N + a


def _indices(adj2):
    nnb = N // BN
    return pl.pallas_call(
        _idx_body,
        grid=(B, nnb),
        in_specs=[pl.BlockSpec((BN, K), lambda b, nb: (b * nnb + nb, 0))],
        out_specs=pl.BlockSpec((BN, K), lambda b, nb: (b * nnb + nb, 0)),
        out_shape=jax.ShapeDtypeStruct((NODES, K), jnp.int32),
    )(adj2)


def _dup_winners(adj2):
    nk = NODES * K
    iota = jnp.arange(nk, dtype=jnp.int32)
    fidx = (jnp.arange(NODES, dtype=jnp.int32)[:, None] * N + adj2).reshape(-1)
    sk, sv = lax.sort((fidx, iota), dimension=0, is_stable=False, num_keys=1)
    is_last = jnp.concatenate([sk[1:] != sk[:-1],
                               jnp.ones((1,), jnp.bool_)])
    run_last = jnp.flip(lax.cummin(
        jnp.flip(jnp.where(is_last, iota, jnp.int32(nk))), axis=0))
    kwin_sorted = sv[run_last] % K
    return (jnp.zeros((nk,), jnp.int32)
            .at[sv].set(kwin_sorted, unique_indices=True,
                        mode="promise_in_bounds"))


def _sc_body(zf, gidx, lo, cnn, imp,
             gi_v, lo_v, rows_v, y0_v, cnn_v, row_v, sem_g):
    wid = lax.axis_index("s") * NC + lax.axis_index("c")
    base = wid * NPW
    b = base // N
    lanes = lax.iota(jnp.int32, L)
    zeros = jnp.zeros((L,), jnp.float32)

    def chunk_body(ci, _):
        node0 = base + ci * G
        pltpu.sync_copy(gidx.at[pl.ds(node0 * K, G * K)], gi_v)
        pltpu.sync_copy(lo.at[pl.ds(node0 * K, G * K)], lo_v)
        pltpu.sync_copy(zf.at[pl.ds(b * T * N + node0 - b * N, G)], y0_v)
        pltpu.async_copy(zf.at[gi_v], rows_v, sem_g).wait()

        def node_body(i, _):
            for oc in range(C // L):
                sl = pl.ds(oc * L, L)
                acc = y0_v[i, sl]
                for k in range(K):
                    acc = acc + rows_v[i * K + k, sl]
                cnn_v[i, sl] = acc
            for j in range(N // L):
                row_v[pl.ds(j * L, L)] = zeros
            nf = node0 + i
            ncol = nf - b * N
            plsc.store_scatter(row_v, [jnp.full((L,), ncol, jnp.int32)],
                               jnp.full((L,), 1.0, jnp.float32),
                               mask=lanes == 0)
            gi16 = gi_v[pl.ds(i * K, K)]
            a16 = gi16 - ((b * T + 1) * N + lanes * N)
            lo16 = lo_v[pl.ds(i * K, K)]
            v16 = plsc.load_gather(cnn_v, [jnp.full((L,), i, jnp.int32), lo16])
            plsc.store_scatter(row_v, [a16], v16)
            pltpu.sync_copy(row_v, imp.at[nf])
            return 0

        lax.fori_loop(0, G, node_body, 0)
        pltpu.sync_copy(cnn_v, cnn.at[pl.ds(node0, G)])
        return 0

    lax.fori_loop(0, NPW // G, chunk_body, 0)


_sc_call = functools.partial(
    pl.kernel,
    out_type=[jax.ShapeDtypeStruct((NODES, C), jnp.float32),
              jax.ShapeDtypeStruct((NODES, N), jnp.float32)],
    mesh=plsc.VectorSubcoreMesh(core_axis_name="c", subcore_axis_name="s",
                                num_cores=NC, num_subcores=NS),
    compiler_params=pltpu.CompilerParams(needs_layout_passes=False),
    scratch_types=[
        pltpu.VMEM((G * K,), jnp.int32),
        pltpu.VMEM((G * K,), jnp.int32),
        pltpu.VMEM((G * K, C), jnp.float32),
        pltpu.VMEM((G, C), jnp.float32),
        pltpu.VMEM((G, C), jnp.float32),
        pltpu.VMEM((N,), jnp.float32),
        pltpu.SemaphoreType.DMA,
    ],
)(_sc_body)


def kernel(x, adj_mat, W_conv, b_conv, W2, b2):
    adj = adj_mat.astype(jnp.int32)
    x2 = x.reshape(NODES, C)
    acat = jnp.transpose(W_conv, (2, 1, 0)).reshape(T * C, C)
    m_flat, bias = _fold(acat, W2.T, b_conv.reshape(1, C), b2.reshape(1, C))
    zf = _proj(x2, m_flat.reshape(T, C, C), bias)
    gidx = _indices(adj.reshape(NODES, K))
    lo = _dup_winners(adj.reshape(NODES, K))
    cnn, imp = _sc_call(zf, gidx.reshape(-1), lo)
    return cnn.reshape(B, N, C), imp.reshape(B, N, N)


# speedup vs baseline: 3.6981x; 1.1991x over previous
import functools

import jax
import jax.numpy as jnp
from jax import lax
from jax.experimental import pallas as pl
from jax.experimental.pallas import tpu as pltpu
from jax.experimental.pallas import tpu_sc as plsc

B, N, C, K = 8, 2048, 256, 16
T = K + 1
NODES = B * N
NC, NS, L = 2, 16, 16
NW = NC * NS
NPW = NODES // NW
G = 8
BM = 512
BN = 256


def _fold_body(a_ref, w2t_ref, bc_ref, b2_ref, m_ref, bias_ref):
    m_ref[...] = jnp.dot(a_ref[...], w2t_ref[...],
                         preferred_element_type=jnp.float32
                         ).astype(jnp.bfloat16)
    bias_ref[...] = jnp.dot(bc_ref[...], w2t_ref[...],
                            preferred_element_type=jnp.float32) + b2_ref[...]


def _fold(acat, w2t, bc, b2):
    return pl.pallas_call(
        _fold_body,
        out_shape=(jax.ShapeDtypeStruct((T * C, C), jnp.bfloat16),
                   jax.ShapeDtypeStruct((1, C), jnp.float32)),
    )(acat, w2t, bc, b2)


def _proj_body(x_ref, m_ref, bias_ref, z_ref):
    t = pl.program_id(1)
    acc = jnp.dot(x_ref[...], m_ref[0],
                  preferred_element_type=jnp.float32)
    z_ref[...] = jnp.where(t == 0, acc + bias_ref[...], acc)


def _proj(x2, mstack, bias):
    nmb = N // BM
    return pl.pallas_call(
        _proj_body,
        grid=(B, T, nmb),
        in_specs=[
            pl.BlockSpec((BM, C), lambda b, t, mb: (b * nmb + mb, 0)),
            pl.BlockSpec((1, C, C), lambda b, t, mb: (t, 0, 0)),
            pl.BlockSpec((1, C), lambda b, t, mb: (0, 0)),
        ],
        out_specs=pl.BlockSpec(
            (BM, C), lambda b, t, mb: ((b * T + t) * nmb + mb, 0)),
        out_shape=jax.ShapeDtypeStruct((B * T * N, C), jnp.float32),
    )(x2, mstack, bias)


def _idx_body(adj_ref, gidx_ref):
    b = pl.program_id(0)
    a = adj_ref[...]
    k_iota = lax.broadcasted_iota(jnp.int32, (BN, K), 1)
    gidx_ref[...] = (b * T + 1 + k_iota) * N + a


def _indices(adj2):
    nnb = N // BN
    return pl.pallas_call(
        _idx_body,
        grid=(B, nnb),
        in_specs=[pl.BlockSpec((BN, K), lambda b, nb: (b * nnb + nb, 0))],
        out_specs=pl.BlockSpec((BN, K), lambda b, nb: (b * nnb + nb, 0)),
        out_shape=jax.ShapeDtypeStruct((NODES, K), jnp.int32),
    )(adj2)


def _dup_winners(adj2):
    nk = NODES * K
    iota = jnp.arange(nk, dtype=jnp.int32)
    fidx = (jnp.arange(NODES, dtype=jnp.int32)[:, None] * N + adj2).reshape(-1)
    sk, sv = lax.sort((fidx, iota), dimension=0, is_stable=False, num_keys=1)
    is_last = jnp.concatenate([sk[1:] != sk[:-1],
                               jnp.ones((1,), jnp.bool_)])
    def comb(a, b):
        va, fa = a
        vb, fb = b
        return jnp.where(fb, vb, va), fa | fb
    filled, _ = lax.associative_scan(
        comb, (jnp.flip(sv % K), jnp.flip(is_last)))
    wk_sorted = jnp.flip(filled)
    _, kwin = lax.sort((sv, wk_sorted), dimension=0, is_stable=False,
                       num_keys=1)
    return kwin


def _sc_body(zf, gidx, lo, cnn, imp,
             gi_v, lo_v, rows_v, y0_v, cnn_v, row_v, sem_g):
    wid = lax.axis_index("s") * NC + lax.axis_index("c")
    base = wid * NPW
    b = base // N
    lanes = lax.iota(jnp.int32, L)
    zeros = jnp.zeros((L,), jnp.float32)

    def chunk_body(ci, _):
        node0 = base + ci * G
        pltpu.sync_copy(gidx.at[pl.ds(node0 * K, G * K)], gi_v)
        pltpu.sync_copy(lo.at[pl.ds(node0 * K, G * K)], lo_v)
        pltpu.sync_copy(zf.at[pl.ds(b * T * N + node0 - b * N, G)], y0_v)
        pltpu.async_copy(zf.at[gi_v], rows_v, sem_g).wait()

        def node_body(i, _):
            for oc in range(C // L):
                sl = pl.ds(oc * L, L)
                acc = y0_v[i, sl]
                for k in range(K):
                    acc = acc + rows_v[i * K + k, sl]
                cnn_v[i, sl] = acc
            for j in range(N // L):
                row_v[pl.ds(j * L, L)] = zeros
            nf = node0 + i
            ncol = nf - b * N
            plsc.store_scatter(row_v, [jnp.full((L,), ncol, jnp.int32)],
                               jnp.full((L,), 1.0, jnp.float32),
                               mask=lanes == 0)
            gi16 = gi_v[pl.ds(i * K, K)]
            a16 = gi16 - ((b * T + 1) * N + lanes * N)
            lo16 = lo_v[pl.ds(i * K, K)]
            v16 = plsc.load_gather(cnn_v, [jnp.full((L,), i, jnp.int32), lo16])
            plsc.store_scatter(row_v, [a16], v16)
            pltpu.sync_copy(row_v, imp.at[nf])
            return 0

        lax.fori_loop(0, G, node_body, 0)
        pltpu.sync_copy(cnn_v, cnn.at[pl.ds(node0, G)])
        return 0

    lax.fori_loop(0, NPW // G, chunk_body, 0)


_sc_call = functools.partial(
    pl.kernel,
    out_type=[jax.ShapeDtypeStruct((NODES, C), jnp.float32),
              jax.ShapeDtypeStruct((NODES, N), jnp.float32)],
    mesh=plsc.VectorSubcoreMesh(core_axis_name="c", subcore_axis_name="s",
                                num_cores=NC, num_subcores=NS),
    compiler_params=pltpu.CompilerParams(needs_layout_passes=False),
    scratch_types=[
        pltpu.VMEM((G * K,), jnp.int32),
        pltpu.VMEM((G * K,), jnp.int32),
        pltpu.VMEM((G * K, C), jnp.float32),
        pltpu.VMEM((G, C), jnp.float32),
        pltpu.VMEM((G, C), jnp.float32),
        pltpu.VMEM((N,), jnp.float32),
        pltpu.SemaphoreType.DMA,
    ],
)(_sc_body)


def kernel(x, adj_mat, W_conv, b_conv, W2, b2):
    adj = adj_mat.astype(jnp.int32)
    x2 = x.reshape(NODES, C).astype(jnp.bfloat16)
    acat = jnp.transpose(W_conv, (2, 1, 0)).reshape(T * C, C)
    m_flat, bias = _fold(acat, W2.T, b_conv.reshape(1, C), b2.reshape(1, C))
    zf = _proj(x2, m_flat.reshape(T, C, C), bias)
    gidx = _indices(adj.reshape(NODES, K))
    lo = _dup_winners(adj.reshape(NODES, K))
    cnn, imp = _sc_call(zf, gidx.reshape(-1), lo)
    return cnn.reshape(B, N, C), imp.reshape(B, N, N)


# speedup vs baseline: 5.0373x; 1.3621x over previous
import functools

import jax
import jax.numpy as jnp
from jax import lax
from jax.experimental import pallas as pl
from jax.experimental.pallas import tpu as pltpu
from jax.experimental.pallas import tpu_sc as plsc

B, N, C, K = 8, 2048, 256, 16
T = K + 1
NODES = B * N
NC, NS, L = 2, 16, 16
NW = NC * NS
NPW = NODES // NW
G = 8
BM = 512
BN = 256


def _fold_body(a_ref, w2t_ref, bc_ref, b2_ref, m_ref, bias_ref):
    m_ref[...] = jnp.dot(a_ref[...], w2t_ref[...],
                         preferred_element_type=jnp.float32
                         ).astype(jnp.bfloat16)
    bias_ref[...] = jnp.dot(bc_ref[...], w2t_ref[...],
                            preferred_element_type=jnp.float32) + b2_ref[...]


def _fold(acat, w2t, bc, b2):
    return pl.pallas_call(
        _fold_body,
        out_shape=(jax.ShapeDtypeStruct((T * C, C), jnp.bfloat16),
                   jax.ShapeDtypeStruct((1, C), jnp.float32)),
    )(acat, w2t, bc, b2)


def _proj_body(x_ref, m_ref, bias_ref, z_ref):
    t = pl.program_id(1)
    acc = jnp.dot(x_ref[...], m_ref[0],
                  preferred_element_type=jnp.float32)
    z_ref[...] = jnp.where(t == 0, acc + bias_ref[...], acc)


def _proj(x2, mstack, bias):
    nmb = N // BM
    return pl.pallas_call(
        _proj_body,
        grid=(B, T, nmb),
        in_specs=[
            pl.BlockSpec((BM, C), lambda b, t, mb: (b * nmb + mb, 0)),
            pl.BlockSpec((1, C, C), lambda b, t, mb: (t, 0, 0)),
            pl.BlockSpec((1, C), lambda b, t, mb: (0, 0)),
        ],
        out_specs=pl.BlockSpec(
            (BM, C), lambda b, t, mb: ((b * T + t) * nmb + mb, 0)),
        out_shape=jax.ShapeDtypeStruct((B * T * N, C), jnp.float32),
    )(x2, mstack, bias)


def _idx_body(adj_ref, gidx_ref):
    b = pl.program_id(0)
    a = adj_ref[...]
    k_iota = lax.broadcasted_iota(jnp.int32, (BN, K), 1)
    gidx_ref[...] = (b * T + 1 + k_iota) * N + a


def _indices(adj2):
    nnb = N // BN
    return pl.pallas_call(
        _idx_body,
        grid=(B, nnb),
        in_specs=[pl.BlockSpec((BN, K), lambda b, nb: (b * nnb + nb, 0))],
        out_specs=pl.BlockSpec((BN, K), lambda b, nb: (b * nnb + nb, 0)),
        out_shape=jax.ShapeDtypeStruct((NODES, K), jnp.int32),
    )(adj2)


def _dup_winners(adj2):
    nk = NODES * K
    iota = jnp.arange(nk, dtype=jnp.int32)
    fidx = (jnp.arange(NODES, dtype=jnp.int32)[:, None] * N + adj2).reshape(-1)
    sk, sv = lax.sort((fidx, iota), dimension=0, is_stable=False, num_keys=1)
    is_last = jnp.concatenate([sk[1:] != sk[:-1],
                               jnp.ones((1,), jnp.bool_)]).astype(jnp.int32)
    _, win = lax.sort((sv, is_last), dimension=0, is_stable=False,
                      num_keys=1)
    return win


def _sc_body(zf, gidx, lo, cnn, imp,
             gi_v, lo_v, rows_v, y0_v, cnn_v, row_v, sem_g):
    wid = lax.axis_index("s") * NC + lax.axis_index("c")
    base = wid * NPW
    b = base // N
    lanes = lax.iota(jnp.int32, L)
    zeros = jnp.zeros((L,), jnp.float32)

    def chunk_body(ci, _):
        node0 = base + ci * G
        pltpu.sync_copy(gidx.at[pl.ds(node0 * K, G * K)], gi_v)
        pltpu.sync_copy(lo.at[pl.ds(node0 * K, G * K)], lo_v)
        pltpu.sync_copy(zf.at[pl.ds(b * T * N + node0 - b * N, G)], y0_v)
        pltpu.async_copy(zf.at[gi_v], rows_v, sem_g).wait()

        def node_body(i, _):
            for oc in range(C // L):
                sl = pl.ds(oc * L, L)
                acc = y0_v[i, sl]
                for k in range(K):
                    acc = acc + rows_v[i * K + k, sl]
                cnn_v[i, sl] = acc
            for j in range(N // L):
                row_v[pl.ds(j * L, L)] = zeros
            nf = node0 + i
            ncol = nf - b * N
            plsc.store_scatter(row_v, [jnp.full((L,), ncol, jnp.int32)],
                               jnp.full((L,), 1.0, jnp.float32),
                               mask=lanes == 0)
            gi16 = gi_v[pl.ds(i * K, K)]
            a16 = gi16 - ((b * T + 1) * N + lanes * N)
            w16 = lo_v[pl.ds(i * K, K)]
            v16 = cnn_v[i, pl.ds(0, L)]
            plsc.store_scatter(row_v, [a16], v16, mask=w16 != 0)
            pltpu.sync_copy(row_v, imp.at[nf])
            return 0

        lax.fori_loop(0, G, node_body, 0)
        pltpu.sync_copy(cnn_v, cnn.at[pl.ds(node0, G)])
        return 0

    lax.fori_loop(0, NPW // G, chunk_body, 0)


_sc_call = functools.partial(
    pl.kernel,
    out_type=[jax.ShapeDtypeStruct((NODES, C), jnp.float32),
              jax.ShapeDtypeStruct((NODES, N), jnp.float32)],
    mesh=plsc.VectorSubcoreMesh(core_axis_name="c", subcore_axis_name="s",
                                num_cores=NC, num_subcores=NS),
    compiler_params=pltpu.CompilerParams(needs_layout_passes=False),
    scratch_types=[
        pltpu.VMEM((G * K,), jnp.int32),
        pltpu.VMEM((G * K,), jnp.int32),
        pltpu.VMEM((G * K, C), jnp.float32),
        pltpu.VMEM((G, C), jnp.float32),
        pltpu.VMEM((G, C), jnp.float32),
        pltpu.VMEM((N,), jnp.float32),
        pltpu.SemaphoreType.DMA,
    ],
)(_sc_body)


def kernel(x, adj_mat, W_conv, b_conv, W2, b2):
    adj = adj_mat.astype(jnp.int32)
    x2 = x.reshape(NODES, C).astype(jnp.bfloat16)
    acat = jnp.transpose(W_conv, (2, 1, 0)).reshape(T * C, C)
    m_flat, bias = _fold(acat, W2.T, b_conv.reshape(1, C), b2.reshape(1, C))
    zf = _proj(x2, m_flat.reshape(T, C, C), bias)
    gidx = _indices(adj.reshape(NODES, K))
    lo = _dup_winners(adj.reshape(NODES, K))
    cnn, imp = _sc_call(zf, gidx.reshape(-1), lo)
    return cnn.reshape(B, N, C), imp.reshape(B, N, N)


# speedup vs baseline: 6.6338x; 1.3169x over previous
import functools

import jax
import jax.numpy as jnp
from jax import lax
from jax.experimental import pallas as pl
from jax.experimental.pallas import tpu as pltpu
from jax.experimental.pallas import tpu_sc as plsc

B, N, C, K = 8, 2048, 256, 16
T = K + 1
NODES = B * N
NC, NS, L = 2, 16, 16
NW = NC * NS
NPW = NODES // NW
G = 8
BM = 512
BN = 256


def _fold_body(a_ref, w2t_ref, bc_ref, b2_ref, m_ref, bias_ref):
    m_ref[...] = jnp.dot(a_ref[...], w2t_ref[...],
                         preferred_element_type=jnp.float32
                         ).astype(jnp.bfloat16)
    bias_ref[...] = jnp.dot(bc_ref[...], w2t_ref[...],
                            preferred_element_type=jnp.float32) + b2_ref[...]


def _fold(acat, w2t, bc, b2):
    return pl.pallas_call(
        _fold_body,
        out_shape=(jax.ShapeDtypeStruct((T * C, C), jnp.bfloat16),
                   jax.ShapeDtypeStruct((1, C), jnp.float32)),
    )(acat, w2t, bc, b2)


def _proj_body(x_ref, m_ref, bias_ref, z_ref):
    x = x_ref[...]
    for t in range(T):
        r = jnp.dot(x, m_ref[t], preferred_element_type=jnp.float32)
        if t == 0:
            r = r + bias_ref[...]
        z_ref[t] = r


def _proj(x2, mstack, bias):
    nmb = N // BM
    return pl.pallas_call(
        _proj_body,
        grid=(B, nmb),
        in_specs=[
            pl.BlockSpec((BM, C), lambda b, mb: (b * nmb + mb, 0)),
            pl.BlockSpec((T, C, C), lambda b, mb: (0, 0, 0)),
            pl.BlockSpec((1, C), lambda b, mb: (0, 0)),
        ],
        out_specs=pl.BlockSpec((T, BM, C), lambda b, mb: (b, mb, 0)),
        out_shape=jax.ShapeDtypeStruct((B * T, N, C), jnp.float32),
    )(x2, mstack, bias)


def _idx_body(adj_ref, gidx_ref):
    b = pl.program_id(0)
    a = adj_ref[...]
    k_iota = lax.broadcasted_iota(jnp.int32, (BN, K), 1)
    gidx_ref[...] = (b * T + 1 + k_iota) * N + a


def _indices(adj2):
    nnb = N // BN
    return pl.pallas_call(
        _idx_body,
        grid=(B, nnb),
        in_specs=[pl.BlockSpec((BN, K), lambda b, nb: (b * nnb + nb, 0))],
        out_specs=pl.BlockSpec((BN, K), lambda b, nb: (b * nnb + nb, 0)),
        out_shape=jax.ShapeDtypeStruct((NODES, K), jnp.int32),
    )(adj2)


def _dup_winners(adj2):
    nk = NODES * K
    iota = jnp.arange(nk, dtype=jnp.int32)
    fidx = (jnp.arange(NODES, dtype=jnp.int32)[:, None] * N + adj2).reshape(-1)
    sk, sv = lax.sort((fidx, iota), dimension=0, is_stable=False, num_keys=1)
    is_last = jnp.concatenate([sk[1:] != sk[:-1],
                               jnp.ones((1,), jnp.bool_)]).astype(jnp.int32)
    _, win = lax.sort((sv, is_last), dimension=0, is_stable=False,
                      num_keys=1)
    return win


def _sc_body(zf, gidx, lo, cnn, imp,
             gi_v, lo_v, rows_v, y0_v, cnn_v, row_v, sem_g):
    wid = lax.axis_index("s") * NC + lax.axis_index("c")
    base = wid * NPW
    b = base // N
    lanes = lax.iota(jnp.int32, L)
    zeros = jnp.zeros((L,), jnp.float32)

    def chunk_body(ci, _):
        node0 = base + ci * G
        pltpu.sync_copy(gidx.at[pl.ds(node0 * K, G * K)], gi_v)
        pltpu.sync_copy(lo.at[pl.ds(node0 * K, G * K)], lo_v)
        pltpu.sync_copy(zf.at[pl.ds(b * T * N + node0 - b * N, G)], y0_v)
        pltpu.async_copy(zf.at[gi_v], rows_v, sem_g).wait()

        def node_body(i, _):
            for oc in range(C // L):
                sl = pl.ds(oc * L, L)
                acc = y0_v[i, sl]
                for k in range(K):
                    acc = acc + rows_v[i * K + k, sl]
                cnn_v[i, sl] = acc
            for j in range(N // L):
                row_v[pl.ds(j * L, L)] = zeros
            nf = node0 + i
            ncol = nf - b * N
            plsc.store_scatter(row_v, [jnp.full((L,), ncol, jnp.int32)],
                               jnp.full((L,), 1.0, jnp.float32),
                               mask=lanes == 0)
            gi16 = gi_v[pl.ds(i * K, K)]
            a16 = gi16 - ((b * T + 1) * N + lanes * N)
            w16 = lo_v[pl.ds(i * K, K)]
            v16 = cnn_v[i, pl.ds(0, L)]
            plsc.store_scatter(row_v, [a16], v16, mask=w16 != 0)
            pltpu.sync_copy(row_v, imp.at[nf])
            return 0

        lax.fori_loop(0, G, node_body, 0)
        pltpu.sync_copy(cnn_v, cnn.at[pl.ds(node0, G)])
        return 0

    lax.fori_loop(0, NPW // G, chunk_body, 0)


_sc_call = functools.partial(
    pl.kernel,
    out_type=[jax.ShapeDtypeStruct((NODES, C), jnp.float32),
              jax.ShapeDtypeStruct((NODES, N), jnp.float32)],
    mesh=plsc.VectorSubcoreMesh(core_axis_name="c", subcore_axis_name="s",
                                num_cores=NC, num_subcores=NS),
    compiler_params=pltpu.CompilerParams(needs_layout_passes=False),
    scratch_types=[
        pltpu.VMEM((G * K,), jnp.int32),
        pltpu.VMEM((G * K,), jnp.int32),
        pltpu.VMEM((G * K, C), jnp.float32),
        pltpu.VMEM((G, C), jnp.float32),
        pltpu.VMEM((G, C), jnp.float32),
        pltpu.VMEM((N,), jnp.float32),
        pltpu.SemaphoreType.DMA,
    ],
)(_sc_body)


def kernel(x, adj_mat, W_conv, b_conv, W2, b2):
    adj = adj_mat.astype(jnp.int32)
    x2 = x.reshape(NODES, C).astype(jnp.bfloat16)
    acat = jnp.transpose(W_conv, (2, 1, 0)).reshape(T * C, C)
    m_flat, bias = _fold(acat, W2.T, b_conv.reshape(1, C), b2.reshape(1, C))
    zf = _proj(x2, m_flat.reshape(T, C, C), bias).reshape(B * T * N, C)
    gidx = _indices(adj.reshape(NODES, K))
    lo = _dup_winners(adj.reshape(NODES, K))
    cnn, imp = _sc_call(zf, gidx.reshape(-1), lo)
    return cnn.reshape(B, N, C), imp.reshape(B, N, N)
